# baseline (speedup 1.0000x reference)
import jax
import jax.numpy as jnp
from jax import lax
from jax.experimental import pallas as pl
from jax.experimental.pallas import tpu as pltpu
from jax.experimental.pallas import tpu_sc as plsc

N = 10000
E = 160000
D = 256
NS = 16
NP = 10240
EPW = E // NS
NPW = NP // NS
NPW_LAST = N - 15 * NPW
L = 16
GRID = 1
YB = NP // GRID
EB = E


def _prep_tc(x_ref, w_ref, coeffs_ref, b_ref, ei_ref,
             oy_ref, os_ref, osrc_ref, odst_ref):
    y = lax.dot_general(w_ref[...], x_ref[...], (((1,), (1,)), ((), ())),
                        preferred_element_type=jnp.float32)
    oy_ref[...] = y[0]

    osrc_ref[...] = ei_ref[0]
    odst_ref[...] = ei_ref[1]

    @pl.when(pl.program_id(0) == 0)
    def _():
        kc = coeffs_ref.shape[0] - 1
        p = [1.0, 0.0]
        q = [0.0, 1.0]
        for _ in range(2, kc + 1):
            p.append(2.0 * p[-1] - p[-2])
            q.append(2.0 * q[-1] - q[-2])
        alpha = jax.nn.softmax(coeffs_ref[...], axis=0)
        c_x = sum(p[k] * alpha[k] for k in range(kc + 1))
        c_a = sum(q[k] * alpha[k] for k in range(kc + 1))
        bb = b_ref[0]
        os_ref[...] = jnp.concatenate([
            jnp.full((L,), c_x, jnp.float32),
            jnp.full((L,), c_a, jnp.float32),
            jnp.full((L,), bb, jnp.float32),
        ])


def _rsqrt16(d):
    i = plsc.bitcast(d, jnp.int32)
    i = jnp.int32(0x5F3759DF) - lax.shift_right_logical(i, jnp.int32(1))
    r = plsc.bitcast(i, jnp.float32)
    for _ in range(3):
        r = r * (1.5 - 0.5 * d * r * r)
    return r


def _sc_body(src_hbm, dst_hbm, y_hbm, scal_hbm, out_hbm,
             src_v, dst_v, vals_v, gat_v, ch_y, ch_n, ch_a, ch_t, scal_v,
             sh_acc, sh_s, sem):
    w = lax.axis_index("s")
    ebase = w * EPW
    nbase = w * NPW

    c1 = pltpu.async_copy(src_hbm.at[pl.ds(ebase, EPW)], src_v, sem)
    c2 = pltpu.async_copy(dst_hbm.at[pl.ds(ebase, EPW)], dst_v, sem)
    c3 = pltpu.async_copy(y_hbm.at[pl.ds(nbase, NPW)], ch_y, sem)
    c4 = pltpu.async_copy(scal_hbm, scal_v, sem)

    ones = jnp.full((L,), 1.0, jnp.float32)
    zeros = jnp.zeros((L,), jnp.float32)

    def zero_ref(ref, base, nvec):
        @plsc.parallel_loop(0, nvec, unroll=4)
        def _(i):
            ref[pl.ds(base + i * L, L)] = zeros

    zero_ref(ch_t, 0, NPW // L)
    pltpu.sync_copy(ch_t, sh_acc.at[pl.ds(nbase, NPW)])
    c1.wait()
    c2.wait()
    c3.wait()
    c4.wait()

    @pl.when(w == NS - 1)
    def _():
        zero_ref(ch_y, NPW_LAST, (NPW - NPW_LAST) // L)

    @plsc.parallel_loop(0, EPW // L, unroll=8)
    def _(e):
        ds = pl.ds(e * L, L)
        s = src_v[ds]
        m = s != dst_v[ds]
        vals_v[ds] = jnp.where(m, ones, zeros)
        src_v[ds] = jnp.where(m, s, jnp.full((L,), NP - 1, jnp.int32))

    plsc.subcore_barrier()
    pltpu.sync_copy(vals_v, sh_acc.at[dst_v], add=True)
    plsc.subcore_barrier()
    pltpu.sync_copy(sh_acc.at[pl.ds(nbase, NPW)], ch_a)

    @plsc.parallel_loop(0, NPW // L, unroll=4)
    def _(j):
        ds = pl.ds(j * L, L)
        r = _rsqrt16(ch_a[ds] + 1.0)
        ch_n[ds] = r
        ch_t[ds] = ch_y[ds] * r

    pltpu.sync_copy(ch_t, sh_s.at[pl.ds(nbase, NPW)])

    zero_ref(ch_t, 0, NPW // L)
    pltpu.sync_copy(ch_t, sh_acc.at[pl.ds(nbase, NPW)])
    plsc.subcore_barrier()

    pltpu.sync_copy(sh_s.at[src_v], gat_v)
    pltpu.sync_copy(gat_v, sh_acc.at[dst_v], add=True)
    plsc.subcore_barrier()
    pltpu.sync_copy(sh_acc.at[pl.ds(nbase, NPW)], ch_a)

    cx = scal_v[pl.ds(0, L)]
    ca = scal_v[pl.ds(L, L)]
    bb = scal_v[pl.ds(2 * L, L)]

    @plsc.parallel_loop(0, NPW // L, unroll=4)
    def _(j):
        ds = pl.ds(j * L, L)
        s_me = ch_y[ds] * ch_n[ds]
        ch_t[ds] = cx * ch_y[ds] + ca * (ch_n[ds] * (ch_a[ds] + s_me)) + bb

    @pl.when(w < NS - 1)
    def _():
        pltpu.sync_copy(ch_t, out_hbm.at[pl.ds(nbase, NPW)])

    @pl.when(w == NS - 1)
    def _():
        pltpu.sync_copy(ch_t.at[pl.ds(0, NPW_LAST)],
                        out_hbm.at[pl.ds((NS - 1) * NPW, NPW_LAST)])


@jax.jit
def kernel(x, edge_index, coeffs, W, b):
    yv, scal, srcv, dstv = pl.pallas_call(
        _prep_tc,
        grid=(GRID,),
        in_specs=[
            pl.BlockSpec((YB, D), lambda i: (i, 0)),
            pl.BlockSpec((1, D), lambda i: (0, 0)),
            pl.BlockSpec((coeffs.shape[0],), lambda i: (0,)),
            pl.BlockSpec((1,), lambda i: (0,)),
            pl.BlockSpec((2, EB), lambda i: (0, i)),
        ],
        out_specs=(
            pl.BlockSpec((YB,), lambda i: (i,)),
            pl.BlockSpec((3 * L,), lambda i: (0,)),
            pl.BlockSpec((EB,), lambda i: (i,)),
            pl.BlockSpec((EB,), lambda i: (i,)),
        ),
        out_shape=(jax.ShapeDtypeStruct((NP,), jnp.float32),
                   jax.ShapeDtypeStruct((3 * L,), jnp.float32),
                   jax.ShapeDtypeStruct((E,), jnp.int32),
                   jax.ShapeDtypeStruct((E,), jnp.int32)),
    )(x, W, coeffs, b, edge_index)

    mesh = plsc.VectorSubcoreMesh(
        core_axis_name="c", subcore_axis_name="s", num_cores=1,
        num_subcores=NS)
    sc = pl.kernel(
        _sc_body,
        out_type=jax.ShapeDtypeStruct((N,), jnp.float32),
        mesh=mesh,
        compiler_params=pltpu.CompilerParams(needs_layout_passes=False),
        scratch_types=[
            pltpu.VMEM((EPW,), jnp.int32),
            pltpu.VMEM((EPW,), jnp.int32),
            pltpu.VMEM((EPW,), jnp.float32),
            pltpu.VMEM((EPW,), jnp.float32),
            pltpu.VMEM((NPW,), jnp.float32),
            pltpu.VMEM((NPW,), jnp.float32),
            pltpu.VMEM((NPW,), jnp.float32),
            pltpu.VMEM((NPW,), jnp.float32),
            pltpu.VMEM((3 * L,), jnp.float32),
            pltpu.VMEM_SHARED((NP,), jnp.float32),
            pltpu.VMEM_SHARED((NP,), jnp.float32),
            pltpu.SemaphoreType.DMA,
        ],
    )
    return sc(srcv, dstv, yv, scal)


# speedup vs baseline: 1.0228x; 1.0228x over previous
import jax
import jax.numpy as jnp
from jax import lax
from jax.experimental import pallas as pl
from jax.experimental.pallas import tpu as pltpu
from jax.experimental.pallas import tpu_sc as plsc

N = 10000
E = 160000
D = 256
NS = 16
NP = 10240
EPW = E // NS
NPW = NP // NS
NPW_LAST = N - 15 * NPW
L = 16
GRID = 2
YB = NP // GRID
EB = 81920


def _prep_tc(x_ref, w_ref, coeffs_ref, b_ref, ei_ref,
             oy_ref, os_ref, osrc_ref, odst_ref):
    y = lax.dot_general(w_ref[...], x_ref[...], (((1,), (1,)), ((), ())),
                        preferred_element_type=jnp.float32)
    oy_ref[...] = y[0]

    osrc_ref[...] = ei_ref[0]
    odst_ref[...] = ei_ref[1]

    @pl.when(pl.program_id(0) == 0)
    def _():
        kc = coeffs_ref.shape[0] - 1
        p = [1.0, 0.0]
        q = [0.0, 1.0]
        for _ in range(2, kc + 1):
            p.append(2.0 * p[-1] - p[-2])
            q.append(2.0 * q[-1] - q[-2])
        alpha = jax.nn.softmax(coeffs_ref[...], axis=0)
        c_x = sum(p[k] * alpha[k] for k in range(kc + 1))
        c_a = sum(q[k] * alpha[k] for k in range(kc + 1))
        bb = b_ref[0]
        os_ref[...] = jnp.concatenate([
            jnp.full((L,), c_x, jnp.float32),
            jnp.full((L,), c_a, jnp.float32),
            jnp.full((L,), bb, jnp.float32),
        ])


def _rsqrt16(d):
    i = plsc.bitcast(d, jnp.int32)
    i = jnp.int32(0x5F3759DF) - lax.shift_right_logical(i, jnp.int32(1))
    r = plsc.bitcast(i, jnp.float32)
    for _ in range(3):
        r = r * (1.5 - 0.5 * d * r * r)
    return r


def _sc_body(src_hbm, dst_hbm, y_hbm, scal_hbm, out_hbm,
             src_v, dst_v, vals_v, gat_v, ch_y, ch_n, ch_a, ch_t, scal_v,
             sh_acc, sh_s, sem):
    w = lax.axis_index("s")
    ebase = w * EPW
    nbase = w * NPW

    c1 = pltpu.async_copy(src_hbm.at[pl.ds(ebase, EPW)], src_v, sem)
    c2 = pltpu.async_copy(dst_hbm.at[pl.ds(ebase, EPW)], dst_v, sem)
    c3 = pltpu.async_copy(y_hbm.at[pl.ds(nbase, NPW)], ch_y, sem)
    c4 = pltpu.async_copy(scal_hbm, scal_v, sem)

    ones = jnp.full((L,), 1.0, jnp.float32)
    zeros = jnp.zeros((L,), jnp.float32)

    def zero_ref(ref, base, nvec):
        @plsc.parallel_loop(0, nvec, unroll=4)
        def _(i):
            ref[pl.ds(base + i * L, L)] = zeros

    zero_ref(ch_t, 0, NPW // L)
    pltpu.sync_copy(ch_t, sh_acc.at[pl.ds(nbase, NPW)])
    c1.wait()
    c2.wait()
    c3.wait()
    c4.wait()

    @pl.when(w == NS - 1)
    def _():
        zero_ref(ch_y, NPW_LAST, (NPW - NPW_LAST) // L)

    @plsc.parallel_loop(0, EPW // L, unroll=8)
    def _(e):
        ds = pl.ds(e * L, L)
        s = src_v[ds]
        m = s != dst_v[ds]
        vals_v[ds] = jnp.where(m, ones, zeros)
        src_v[ds] = jnp.where(m, s, jnp.full((L,), NP - 1, jnp.int32))

    plsc.subcore_barrier()
    pltpu.sync_copy(vals_v, sh_acc.at[dst_v], add=True)
    plsc.subcore_barrier()
    pltpu.sync_copy(sh_acc.at[pl.ds(nbase, NPW)], ch_a)

    @plsc.parallel_loop(0, NPW // L, unroll=4)
    def _(j):
        ds = pl.ds(j * L, L)
        r = _rsqrt16(ch_a[ds] + 1.0)
        ch_n[ds] = r
        ch_t[ds] = ch_y[ds] * r

    pltpu.sync_copy(ch_t, sh_s.at[pl.ds(nbase, NPW)])

    zero_ref(ch_t, 0, NPW // L)
    pltpu.sync_copy(ch_t, sh_acc.at[pl.ds(nbase, NPW)])
    plsc.subcore_barrier()

    pltpu.sync_copy(sh_s.at[src_v], gat_v)
    pltpu.sync_copy(gat_v, sh_acc.at[dst_v], add=True)
    plsc.subcore_barrier()
    pltpu.sync_copy(sh_acc.at[pl.ds(nbase, NPW)], ch_a)

    cx = scal_v[pl.ds(0, L)]
    ca = scal_v[pl.ds(L, L)]
    bb = scal_v[pl.ds(2 * L, L)]

    @plsc.parallel_loop(0, NPW // L, unroll=4)
    def _(j):
        ds = pl.ds(j * L, L)
        s_me = ch_y[ds] * ch_n[ds]
        ch_t[ds] = cx * ch_y[ds] + ca * (ch_n[ds] * (ch_a[ds] + s_me)) + bb

    @pl.when(w < NS - 1)
    def _():
        pltpu.sync_copy(ch_t, out_hbm.at[pl.ds(nbase, NPW)])

    @pl.when(w == NS - 1)
    def _():
        pltpu.sync_copy(ch_t.at[pl.ds(0, NPW_LAST)],
                        out_hbm.at[pl.ds((NS - 1) * NPW, NPW_LAST)])


@jax.jit
def kernel(x, edge_index, coeffs, W, b):
    yv, scal, srcv, dstv = pl.pallas_call(
        _prep_tc,
        grid=(GRID,),
        in_specs=[
            pl.BlockSpec((YB, D), lambda i: (i, 0)),
            pl.BlockSpec((1, D), lambda i: (0, 0)),
            pl.BlockSpec((coeffs.shape[0],), lambda i: (0,)),
            pl.BlockSpec((1,), lambda i: (0,)),
            pl.BlockSpec((2, EB), lambda i: (0, i)),
        ],
        out_specs=(
            pl.BlockSpec((YB,), lambda i: (i,)),
            pl.BlockSpec((3 * L,), lambda i: (0,)),
            pl.BlockSpec((EB,), lambda i: (i,)),
            pl.BlockSpec((EB,), lambda i: (i,)),
        ),
        out_shape=(jax.ShapeDtypeStruct((NP,), jnp.float32),
                   jax.ShapeDtypeStruct((3 * L,), jnp.float32),
                   jax.ShapeDtypeStruct((E,), jnp.int32),
                   jax.ShapeDtypeStruct((E,), jnp.int32)),
    )(x, W, coeffs, b, edge_index)

    mesh = plsc.VectorSubcoreMesh(
        core_axis_name="c", subcore_axis_name="s", num_cores=1,
        num_subcores=NS)
    sc = pl.kernel(
        _sc_body,
        out_type=jax.ShapeDtypeStruct((N,), jnp.float32),
        mesh=mesh,
        compiler_params=pltpu.CompilerParams(needs_layout_passes=False),
        scratch_types=[
            pltpu.VMEM((EPW,), jnp.int32),
            pltpu.VMEM((EPW,), jnp.int32),
            pltpu.VMEM((EPW,), jnp.float32),
            pltpu.VMEM((EPW,), jnp.float32),
            pltpu.VMEM((NPW,), jnp.float32),
            pltpu.VMEM((NPW,), jnp.float32),
            pltpu.VMEM((NPW,), jnp.float32),
            pltpu.VMEM((NPW,), jnp.float32),
            pltpu.VMEM((3 * L,), jnp.float32),
            pltpu.VMEM_SHARED((NP,), jnp.float32),
            pltpu.VMEM_SHARED((NP,), jnp.float32),
            pltpu.SemaphoreType.DMA,
        ],
    )
    return sc(srcv, dstv, yv, scal)


# speedup vs baseline: 1.0295x; 1.0066x over previous
import jax
import jax.numpy as jnp
from jax import lax
from jax.experimental import pallas as pl
from jax.experimental.pallas import tpu as pltpu
from jax.experimental.pallas import tpu_sc as plsc

N = 10000
E = 160000
D = 256
NS = 16
NP = 10240
EPW = E // NS
NPW = NP // NS
NPW_LAST = N - 15 * NPW
L = 16
GRID = 2
YB = NP // GRID
EB = 81920


def _prep_tc(x_ref, w_ref, coeffs_ref, b_ref, ei_ref,
             oy_ref, os_ref, osrc_ref, odst_ref):
    y = lax.dot_general(w_ref[...], x_ref[...], (((1,), (1,)), ((), ())),
                        preferred_element_type=jnp.float32)
    oy_ref[...] = y[0]

    osrc_ref[...] = ei_ref[0]
    odst_ref[...] = ei_ref[1]

    @pl.when(pl.program_id(0) == 0)
    def _():
        kc = coeffs_ref.shape[0] - 1
        p = [1.0, 0.0]
        q = [0.0, 1.0]
        for _ in range(2, kc + 1):
            p.append(2.0 * p[-1] - p[-2])
            q.append(2.0 * q[-1] - q[-2])
        alpha = jax.nn.softmax(coeffs_ref[...], axis=0)
        c_x = sum(p[k] * alpha[k] for k in range(kc + 1))
        c_a = sum(q[k] * alpha[k] for k in range(kc + 1))
        bb = b_ref[0]
        os_ref[...] = jnp.concatenate([
            jnp.full((L,), c_x, jnp.float32),
            jnp.full((L,), c_a, jnp.float32),
            jnp.full((L,), bb, jnp.float32),
        ])


def _rsqrt16(d):
    i = plsc.bitcast(d, jnp.int32)
    i = jnp.int32(0x5F3759DF) - lax.shift_right_logical(i, jnp.int32(1))
    r = plsc.bitcast(i, jnp.float32)
    for _ in range(3):
        r = r * (1.5 - 0.5 * d * r * r)
    return r


def _sc_body(src_hbm, dst_hbm, y_hbm, scal_hbm, out_hbm,
             src_v, dst_v, vals_v, gat_v, ch_y, ch_n, ch_a, ch_t, scal_v,
             sh_acc, sh_s, sem_a, sem_b, sem_c):
    w = lax.axis_index("s")
    ebase = w * EPW
    nbase = w * NPW

    half = EPW // 2
    c1a = pltpu.async_copy(src_hbm.at[pl.ds(ebase, half)],
                           src_v.at[pl.ds(0, half)], sem_a)
    c2a = pltpu.async_copy(dst_hbm.at[pl.ds(ebase, half)],
                           dst_v.at[pl.ds(0, half)], sem_a)
    c1b = pltpu.async_copy(src_hbm.at[pl.ds(ebase + half, half)],
                           src_v.at[pl.ds(half, half)], sem_b)
    c2b = pltpu.async_copy(dst_hbm.at[pl.ds(ebase + half, half)],
                           dst_v.at[pl.ds(half, half)], sem_b)
    c3 = pltpu.async_copy(y_hbm.at[pl.ds(nbase, NPW)], ch_y, sem_c)
    c4 = pltpu.async_copy(scal_hbm, scal_v, sem_c)

    ones = jnp.full((L,), 1.0, jnp.float32)
    zeros = jnp.zeros((L,), jnp.float32)

    def zero_ref(ref, base, nvec):
        @plsc.parallel_loop(0, nvec, unroll=4)
        def _(i):
            ref[pl.ds(base + i * L, L)] = zeros

    zero_ref(ch_t, 0, NPW // L)
    pltpu.sync_copy(ch_t, sh_acc.at[pl.ds(nbase, NPW)])

    c1a.wait()
    c2a.wait()
    c1b.wait()
    c2b.wait()
    c3.wait()
    c4.wait()

    @plsc.parallel_loop(0, EPW // L, unroll=8)
    def _(e):
        ds = pl.ds(e * L, L)
        s = src_v[ds]
        m = s != dst_v[ds]
        vals_v[ds] = jnp.where(m, ones, zeros)
        src_v[ds] = jnp.where(m, s, jnp.full((L,), NP - 1, jnp.int32))

    @pl.when(w == NS - 1)
    def _():
        zero_ref(ch_y, NPW_LAST, (NPW - NPW_LAST) // L)

    plsc.subcore_barrier()
    pltpu.sync_copy(vals_v, sh_acc.at[dst_v], add=True)
    plsc.subcore_barrier()
    pltpu.sync_copy(sh_acc.at[pl.ds(nbase, NPW)], ch_a)

    @plsc.parallel_loop(0, NPW // L, unroll=4)
    def _(j):
        ds = pl.ds(j * L, L)
        r = _rsqrt16(ch_a[ds] + 1.0)
        ch_n[ds] = r
        ch_t[ds] = ch_y[ds] * r

    pltpu.sync_copy(ch_t, sh_s.at[pl.ds(nbase, NPW)])

    zero_ref(ch_t, 0, NPW // L)
    pltpu.sync_copy(ch_t, sh_acc.at[pl.ds(nbase, NPW)])
    plsc.subcore_barrier()

    pltpu.sync_copy(sh_s.at[src_v], gat_v)
    pltpu.sync_copy(gat_v, sh_acc.at[dst_v], add=True)
    plsc.subcore_barrier()
    pltpu.sync_copy(sh_acc.at[pl.ds(nbase, NPW)], ch_a)

    cx = scal_v[pl.ds(0, L)]
    ca = scal_v[pl.ds(L, L)]
    bb = scal_v[pl.ds(2 * L, L)]

    @plsc.parallel_loop(0, NPW // L, unroll=4)
    def _(j):
        ds = pl.ds(j * L, L)
        s_me = ch_y[ds] * ch_n[ds]
        ch_t[ds] = cx * ch_y[ds] + ca * (ch_n[ds] * (ch_a[ds] + s_me)) + bb

    @pl.when(w < NS - 1)
    def _():
        pltpu.sync_copy(ch_t, out_hbm.at[pl.ds(nbase, NPW)])

    @pl.when(w == NS - 1)
    def _():
        pltpu.sync_copy(ch_t.at[pl.ds(0, NPW_LAST)],
                        out_hbm.at[pl.ds((NS - 1) * NPW, NPW_LAST)])


@jax.jit
def kernel(x, edge_index, coeffs, W, b):
    yv, scal, srcv, dstv = pl.pallas_call(
        _prep_tc,
        grid=(GRID,),
        in_specs=[
            pl.BlockSpec((YB, D), lambda i: (i, 0)),
            pl.BlockSpec((1, D), lambda i: (0, 0)),
            pl.BlockSpec((coeffs.shape[0],), lambda i: (0,)),
            pl.BlockSpec((1,), lambda i: (0,)),
            pl.BlockSpec((2, EB), lambda i: (0, i)),
        ],
        out_specs=(
            pl.BlockSpec((YB,), lambda i: (i,)),
            pl.BlockSpec((3 * L,), lambda i: (0,)),
            pl.BlockSpec((EB,), lambda i: (i,)),
            pl.BlockSpec((EB,), lambda i: (i,)),
        ),
        out_shape=(jax.ShapeDtypeStruct((NP,), jnp.float32),
                   jax.ShapeDtypeStruct((3 * L,), jnp.float32),
                   jax.ShapeDtypeStruct((E,), jnp.int32),
                   jax.ShapeDtypeStruct((E,), jnp.int32)),
    )(x, W, coeffs, b, edge_index)

    mesh = plsc.VectorSubcoreMesh(
        core_axis_name="c", subcore_axis_name="s", num_cores=1,
        num_subcores=NS)
    sc = pl.kernel(
        _sc_body,
        out_type=jax.ShapeDtypeStruct((N,), jnp.float32),
        mesh=mesh,
        compiler_params=pltpu.CompilerParams(needs_layout_passes=False),
        scratch_types=[
            pltpu.VMEM((EPW,), jnp.int32),
            pltpu.VMEM((EPW,), jnp.int32),
            pltpu.VMEM((EPW,), jnp.float32),
            pltpu.VMEM((EPW,), jnp.float32),
            pltpu.VMEM((NPW,), jnp.float32),
            pltpu.VMEM((NPW,), jnp.float32),
            pltpu.VMEM((NPW,), jnp.float32),
            pltpu.VMEM((NPW,), jnp.float32),
            pltpu.VMEM((3 * L,), jnp.float32),
            pltpu.VMEM_SHARED((NP,), jnp.float32),
            pltpu.VMEM_SHARED((NP,), jnp.float32),
            pltpu.SemaphoreType.DMA,
            pltpu.SemaphoreType.DMA,
            pltpu.SemaphoreType.DMA,
        ],
    )
    return sc(srcv, dstv, yv, scal)


# speedup vs baseline: 1.0298x; 1.0002x over previous
import jax
import jax.numpy as jnp
from jax import lax
from jax.experimental import pallas as pl
from jax.experimental.pallas import tpu as pltpu
from jax.experimental.pallas import tpu_sc as plsc

N = 10000
E = 160000
D = 256
NS = 16
NP = 10240
EPW = E // NS
NPW = NP // NS
NPW_LAST = N - 15 * NPW
L = 16
GRID = 2
YB = NP // GRID
EB = 81920


def _prep_tc(x_ref, w_ref, coeffs_ref, b_ref, ei_ref,
             oy_ref, os_ref, osrc_ref, odst_ref):
    y = lax.dot_general(w_ref[...], x_ref[...], (((1,), (1,)), ((), ())),
                        preferred_element_type=jnp.float32)
    oy_ref[...] = y[0]

    osrc_ref[...] = ei_ref[0]
    odst_ref[...] = ei_ref[1]

    @pl.when(pl.program_id(0) == 0)
    def _():
        kc = coeffs_ref.shape[0] - 1
        p = [1.0, 0.0]
        q = [0.0, 1.0]
        for _ in range(2, kc + 1):
            p.append(2.0 * p[-1] - p[-2])
            q.append(2.0 * q[-1] - q[-2])
        alpha = jax.nn.softmax(coeffs_ref[...], axis=0)
        c_x = sum(p[k] * alpha[k] for k in range(kc + 1))
        c_a = sum(q[k] * alpha[k] for k in range(kc + 1))
        bb = b_ref[0]
        os_ref[...] = jnp.concatenate([
            jnp.full((L,), c_x, jnp.float32),
            jnp.full((L,), c_a, jnp.float32),
            jnp.full((L,), bb, jnp.float32),
        ])


def _rsqrt16(d):
    i = plsc.bitcast(d, jnp.int32)
    i = jnp.int32(0x5F3759DF) - lax.shift_right_logical(i, jnp.int32(1))
    r = plsc.bitcast(i, jnp.float32)
    for _ in range(3):
        r = r * (1.5 - 0.5 * d * r * r)
    return r


def _sc_body(src_hbm, dst_hbm, y_hbm, scal_hbm, out_hbm,
             src_v, dst_v, vals_v, gat_v, ch_y, ch_n, ch_a, ch_t, scal_v,
             sh_acc, sh_s, sem_a, sem_b, sem_c):
    w = lax.axis_index("s")
    ebase = w * EPW
    nbase = w * NPW

    half = EPW // 2
    c1a = pltpu.async_copy(src_hbm.at[pl.ds(ebase, half)],
                           src_v.at[pl.ds(0, half)], sem_a)
    c2a = pltpu.async_copy(dst_hbm.at[pl.ds(ebase, half)],
                           dst_v.at[pl.ds(0, half)], sem_a)
    c1b = pltpu.async_copy(src_hbm.at[pl.ds(ebase + half, half)],
                           src_v.at[pl.ds(half, half)], sem_b)
    c2b = pltpu.async_copy(dst_hbm.at[pl.ds(ebase + half, half)],
                           dst_v.at[pl.ds(half, half)], sem_b)
    c3 = pltpu.async_copy(y_hbm.at[pl.ds(nbase, NPW)], ch_y, sem_c)
    c4 = pltpu.async_copy(scal_hbm, scal_v, sem_c)

    ones = jnp.full((L,), 1.0, jnp.float32)
    zeros = jnp.zeros((L,), jnp.float32)

    def zero_ref(ref, base, nvec):
        @plsc.parallel_loop(0, nvec, unroll=8)
        def _(i):
            ref[pl.ds(base + i * L, L)] = zeros

    zero_ref(ch_t, 0, NPW // L)
    pltpu.sync_copy(ch_t, sh_acc.at[pl.ds(nbase, NPW)])

    c1a.wait()
    c2a.wait()
    c1b.wait()
    c2b.wait()
    c3.wait()
    c4.wait()

    @plsc.parallel_loop(0, EPW // L, unroll=16)
    def _(e):
        ds = pl.ds(e * L, L)
        s = src_v[ds]
        m = s != dst_v[ds]
        vals_v[ds] = jnp.where(m, ones, zeros)
        src_v[ds] = jnp.where(m, s, jnp.full((L,), NP - 1, jnp.int32))

    @pl.when(w == NS - 1)
    def _():
        zero_ref(ch_y, NPW_LAST, (NPW - NPW_LAST) // L)

    plsc.subcore_barrier()
    pltpu.sync_copy(vals_v, sh_acc.at[dst_v], add=True)
    plsc.subcore_barrier()
    pltpu.sync_copy(sh_acc.at[pl.ds(nbase, NPW)], ch_a)

    @plsc.parallel_loop(0, NPW // L, unroll=8)
    def _(j):
        ds = pl.ds(j * L, L)
        r = _rsqrt16(ch_a[ds] + 1.0)
        ch_n[ds] = r
        ch_t[ds] = ch_y[ds] * r

    pltpu.sync_copy(ch_t, sh_s.at[pl.ds(nbase, NPW)])

    zero_ref(ch_t, 0, NPW // L)
    pltpu.sync_copy(ch_t, sh_acc.at[pl.ds(nbase, NPW)])
    plsc.subcore_barrier()

    pltpu.sync_copy(sh_s.at[src_v], gat_v)
    pltpu.sync_copy(gat_v, sh_acc.at[dst_v], add=True)
    plsc.subcore_barrier()
    pltpu.sync_copy(sh_acc.at[pl.ds(nbase, NPW)], ch_a)

    cx = scal_v[pl.ds(0, L)]
    ca = scal_v[pl.ds(L, L)]
    bb = scal_v[pl.ds(2 * L, L)]

    @plsc.parallel_loop(0, NPW // L, unroll=8)
    def _(j):
        ds = pl.ds(j * L, L)
        s_me = ch_y[ds] * ch_n[ds]
        ch_t[ds] = cx * ch_y[ds] + ca * (ch_n[ds] * (ch_a[ds] + s_me)) + bb

    @pl.when(w < NS - 1)
    def _():
        pltpu.sync_copy(ch_t, out_hbm.at[pl.ds(nbase, NPW)])

    @pl.when(w == NS - 1)
    def _():
        pltpu.sync_copy(ch_t.at[pl.ds(0, NPW_LAST)],
                        out_hbm.at[pl.ds((NS - 1) * NPW, NPW_LAST)])


@jax.jit
def kernel(x, edge_index, coeffs, W, b):
    yv, scal, srcv, dstv = pl.pallas_call(
        _prep_tc,
        grid=(GRID,),
        in_specs=[
            pl.BlockSpec((YB, D), lambda i: (i, 0)),
            pl.BlockSpec((1, D), lambda i: (0, 0)),
            pl.BlockSpec((coeffs.shape[0],), lambda i: (0,)),
            pl.BlockSpec((1,), lambda i: (0,)),
            pl.BlockSpec((2, EB), lambda i: (0, i)),
        ],
        out_specs=(
            pl.BlockSpec((YB,), lambda i: (i,)),
            pl.BlockSpec((3 * L,), lambda i: (0,)),
            pl.BlockSpec((EB,), lambda i: (i,)),
            pl.BlockSpec((EB,), lambda i: (i,)),
        ),
        out_shape=(jax.ShapeDtypeStruct((NP,), jnp.float32),
                   jax.ShapeDtypeStruct((3 * L,), jnp.float32),
                   jax.ShapeDtypeStruct((E,), jnp.int32),
                   jax.ShapeDtypeStruct((E,), jnp.int32)),
    )(x, W, coeffs, b, edge_index)

    mesh = plsc.VectorSubcoreMesh(
        core_axis_name="c", subcore_axis_name="s", num_cores=1,
        num_subcores=NS)
    sc = pl.kernel(
        _sc_body,
        out_type=jax.ShapeDtypeStruct((N,), jnp.float32),
        mesh=mesh,
        compiler_params=pltpu.CompilerParams(needs_layout_passes=False),
        scratch_types=[
            pltpu.VMEM((EPW,), jnp.int32),
            pltpu.VMEM((EPW,), jnp.int32),
            pltpu.VMEM((EPW,), jnp.float32),
            pltpu.VMEM((EPW,), jnp.float32),
            pltpu.VMEM((NPW,), jnp.float32),
            pltpu.VMEM((NPW,), jnp.float32),
            pltpu.VMEM((NPW,), jnp.float32),
            pltpu.VMEM((NPW,), jnp.float32),
            pltpu.VMEM((3 * L,), jnp.float32),
            pltpu.VMEM_SHARED((NP,), jnp.float32),
            pltpu.VMEM_SHARED((NP,), jnp.float32),
            pltpu.SemaphoreType.DMA,
            pltpu.SemaphoreType.DMA,
            pltpu.SemaphoreType.DMA,
        ],
    )
    return sc(srcv, dstv, yv, scal)
